# baseline (speedup 1.0000x reference)
import functools

import jax
import jax.numpy as jnp
from jax import lax
from jax.experimental import pallas as pl
from jax.experimental.pallas import tpu as pltpu
from jax.experimental.pallas import tpu_sc as plsc

NC = 2
NS = 16
NW = NC * NS

N_GRID = 10000
N_MESH = 10000
E = 160000
D = 256
H = 256
LN_EPS = 1e-5

NSLAB = 5
ES = E // NSLAB
EBLK = 1000


def _round_f32_to_bf16_bits(x):
  b = lax.bitcast_convert_type(x, jnp.int32)
  rounded = b + jnp.int32(0x7FFF) + (lax.shift_right_logical(b, 16) & 1)
  return lax.shift_right_logical(rounded, 16)


def _pack_halves(y):
  lo = _round_f32_to_bf16_bits(y[:, :H // 2])
  hi = _round_f32_to_bf16_bits(y[:, H // 2:])
  return jnp.bitwise_or(lo, lax.shift_left(hi, 16))


def _unpack_halves(g):
  lo_f = lax.bitcast_convert_type(lax.shift_left(g, 16), jnp.float32)
  hi_f = lax.bitcast_convert_type(jnp.bitwise_and(g, jnp.int32(-65536)),
                                  jnp.float32)
  return jnp.concatenate([lo_f, hi_f], axis=-1)


def _proj_body(mesh_ref, grid_ref, wm_ref, wg_ref, pm_ref, pg_ref):
  pm = jnp.dot(mesh_ref[...], wm_ref[...], preferred_element_type=jnp.float32)
  pg = jnp.dot(grid_ref[...], wg_ref[...], preferred_element_type=jnp.float32)
  pm_ref[...] = _pack_halves(pm)
  pg_ref[...] = _pack_halves(pg)


def _project_nodes(mesh_nfeat, grid_nfeat, wm, wg):
  n = mesh_nfeat.shape[0]
  blk = 1000
  grid = (n // blk,)
  row_spec = pl.BlockSpec((blk, D), lambda i: (i, 0))
  w_spec = pl.BlockSpec((D, H), lambda i: (0, 0))
  return pl.pallas_call(
      _proj_body,
      grid=grid,
      in_specs=[row_spec, row_spec, w_spec, w_spec],
      out_specs=[pl.BlockSpec((blk, H // 2), lambda i: (i, 0))] * 2,
      out_shape=[jax.ShapeDtypeStruct((n, H // 2), jnp.int32)] * 2,
  )(mesh_nfeat, grid_nfeat, wm, wg)


def _layernorm_rows(y, g, b):
  m = jnp.mean(y, axis=-1, keepdims=True)
  yc = y - m
  v = jnp.mean(yc * yc, axis=-1, keepdims=True)
  return yc * lax.rsqrt(v + LN_EPS) * g + b


def _edge_body(x_ref, g1_ref, g2_ref, a_ref, b1_ref, w2_ref, b2_ref,
               g_ref, bg_ref, o_ref):
  x = jnp.dot(x_ref[...], a_ref[...], preferred_element_type=jnp.float32)
  x = x + _unpack_halves(g1_ref[...]) + _unpack_halves(g2_ref[...]) + b1_ref[...]
  h = x * jax.nn.sigmoid(x)
  y = jnp.dot(h, w2_ref[...], preferred_element_type=jnp.float32) + b2_ref[...]
  ef = _layernorm_rows(y, g_ref[...], bg_ref[...])
  o_ref[0, :, :] = ef[:, :D // 2]
  o_ref[1, :, :] = ef[:, D // 2:]


def _edge_mlp_slab(m2g_efeat, gath1, gath2, a, b1, w2, b2, g, bg, slab):
  grid = (ES // EBLK,)
  obl = slab * (ES // EBLK)
  w_spec = pl.BlockSpec((D, H), lambda i: (0, 0))
  v_spec = pl.BlockSpec((1, H), lambda i: (0, 0))
  return pl.pallas_call(
      _edge_body,
      grid=grid,
      in_specs=[pl.BlockSpec((EBLK, D), lambda i: (i + obl, 0)),
                pl.BlockSpec((EBLK, H // 2), lambda i: (i, 0)),
                pl.BlockSpec((EBLK, H // 2), lambda i: (i, 0)),
                w_spec, v_spec,
                pl.BlockSpec((H, D), lambda i: (0, 0)), v_spec,
                v_spec, v_spec],
      out_specs=pl.BlockSpec((2, EBLK, D // 2), lambda i: (0, i, 0)),
      out_shape=jax.ShapeDtypeStruct((2, ES, D // 2), jnp.float32),
  )(m2g_efeat, gath1, gath2, a, b1.reshape(1, H), w2, b2.reshape(1, D),
    g.reshape(1, D), bg.reshape(1, D))


def _node_body(a0_ref, a1_ref, a2_ref, a3_ref, a4_ref, gr_ref,
               w1a_ref, w1b_ref, b1_ref, w2_ref, b2_ref,
               g_ref, bg_ref, o_ref):
  agg = (a0_ref[...] + a1_ref[...] + a2_ref[...] + a3_ref[...] + a4_ref[...])
  x = (jnp.dot(agg, w1a_ref[...], preferred_element_type=jnp.float32)
       + jnp.dot(gr_ref[...], w1b_ref[...], preferred_element_type=jnp.float32)
       + b1_ref[...])
  h = x * jax.nn.sigmoid(x)
  y = jnp.dot(h, w2_ref[...], preferred_element_type=jnp.float32) + b2_ref[...]
  o_ref[...] = _layernorm_rows(y, g_ref[...], bg_ref[...]) + gr_ref[...]


def _node_mlp(aggs, grid_nfeat, w1a, w1b, b1, w2, b2, g, bg):
  blk = 1000
  grid = (N_GRID // blk,)
  row_spec = pl.BlockSpec((blk, D), lambda i: (i, 0))
  w_spec = pl.BlockSpec((D, H), lambda i: (0, 0))
  v_spec = pl.BlockSpec((1, H), lambda i: (0, 0))
  return pl.pallas_call(
      _node_body,
      grid=grid,
      in_specs=[row_spec] * NSLAB + [row_spec, w_spec, w_spec, v_spec,
                pl.BlockSpec((H, D), lambda i: (0, 0)), v_spec,
                v_spec, v_spec],
      out_specs=pl.BlockSpec((blk, D), lambda i: (i, 0)),
      out_shape=jax.ShapeDtypeStruct((N_GRID, D), jnp.float32),
  )(*aggs, grid_nfeat, w1a, w1b, b1.reshape(1, H), w2, b2.reshape(1, D),
    g.reshape(1, D), bg.reshape(1, D))


_EPW = ES // NW
_GC = 56
_GN = _EPW // _GC
_GT = _EPW - _GN * _GC
_GCHUNKS = [_GC] * _GN + ([_GT] if _GT else [])


def _gather_body(slab, pm_hbm, pg_hbm, src_hbm, dst_hbm, g1_hbm, g2_hbm,
                 idx_a, idx_b, ra0, rb0, ra1, rb1,
                 sa0, sb0, sa1, sb1, sw0, sw1):
  wid = lax.axis_index("s") * NC + lax.axis_index("c")
  lbase = wid * _EPW
  base = slab * ES + lbase
  pltpu.sync_copy(src_hbm.at[pl.ds(base, _EPW)], idx_a)
  pltpu.sync_copy(dst_hbm.at[pl.ds(base, _EPW)], idx_b)

  bufs = [(ra0, rb0, sa0, sb0, sw0), (ra1, rb1, sa1, sb1, sw1)]
  nchunks = len(_GCHUNKS)

  def issue(i):
    ra, rb, sa, sb, _ = bufs[i % 2]
    off = i * _GC
    n = _GCHUNKS[i]
    ca = pltpu.async_copy(pm_hbm.at[idx_a.at[pl.ds(off, n)]],
                          ra.at[pl.ds(0, n)], sa)
    cb = pltpu.async_copy(pg_hbm.at[idx_b.at[pl.ds(off, n)]],
                          rb.at[pl.ds(0, n)], sb)
    return ca, cb

  def issue_writes(i):
    ra, rb, _, _, sw = bufs[i % 2]
    off = i * _GC
    n = _GCHUNKS[i]
    wa = pltpu.async_copy(ra.at[pl.ds(0, n)],
                          g1_hbm.at[pl.ds(lbase + off, n), :], sw)
    wb = pltpu.async_copy(rb.at[pl.ds(0, n)],
                          g2_hbm.at[pl.ds(lbase + off, n), :], sw)
    return wa, wb

  pend_w = [None, None]
  g = issue(0)
  for i in range(nchunks):
    if i + 1 < nchunks:
      if pend_w[(i + 1) % 2] is not None:
        for wdesc in pend_w[(i + 1) % 2]:
          wdesc.wait()
        pend_w[(i + 1) % 2] = None
      g_next = issue(i + 1)
    for gdesc in g:
      gdesc.wait()
    pend_w[i % 2] = issue_writes(i)
    if i + 1 < nchunks:
      g = g_next
  for p in pend_w:
    if p is not None:
      for wdesc in p:
        wdesc.wait()


def _gather_rows(pm, pg, src, dst, slab):
  mesh = plsc.VectorSubcoreMesh(core_axis_name="c", subcore_axis_name="s")
  f = pl.kernel(
      functools.partial(_gather_body, slab),
      out_type=[jax.ShapeDtypeStruct((ES, H // 2), jnp.int32)] * 2,
      mesh=mesh,
      scratch_types=[
          pltpu.VMEM((_EPW,), jnp.int32),
          pltpu.VMEM((_EPW,), jnp.int32),
          pltpu.VMEM((_GC, H // 2), jnp.int32),
          pltpu.VMEM((_GC, H // 2), jnp.int32),
          pltpu.VMEM((_GC, H // 2), jnp.int32),
          pltpu.VMEM((_GC, H // 2), jnp.int32),
      ] + [pltpu.SemaphoreType.DMA] * 6,
  )
  return f(pm, pg, src, dst)


_CW = D // NC
_EPT = ES // NS
_SCH = 64
_SN = _EPT // _SCH
_ST = _EPT - _SN * _SCH
_SCHUNKS = [_SCH] * _SN + ([_ST] if _ST else [])
_NPAD = 10240
_RPT = _NPAD // NS


def _scatter_body(slab, ef_hbm, dst_hbm, z_hbm, out_hbm,
                  idx0, idx1, idx_t, rows0, rows1, acc, si0, si1, sr0, sr1):
  c = lax.axis_index("c")
  s = lax.axis_index("s")
  col = c * _CW
  sbase = slab * ES
  bufs = [(idx0, rows0, si0, sr0), (idx1, rows1, si1, sr1)]
  nchunks = len(_SCHUNKS)

  def issue(i):
    idx, rows, si, sr = bufs[i % 2]
    off = s * _EPT + i * _SCH
    n = _SCHUNKS[i]
    idx_dst = idx if n == _SCH else idx_t
    ci = pltpu.async_copy(dst_hbm.at[pl.ds(sbase + off, n)], idx_dst, si)
    cr = pltpu.async_copy(ef_hbm.at[c, pl.ds(off, n), :],
                          rows.at[pl.ds(0, n)], sr)
    return ci, cr

  pend = issue(0)
  pltpu.sync_copy(z_hbm, acc.at[pl.ds(s * _RPT, _RPT), :])
  plsc.subcore_barrier()

  for i in range(nchunks):
    if i + 1 < nchunks:
      nxt = issue(i + 1)
    for d in pend:
      d.wait()
    idx, rows, _, _ = bufs[i % 2]
    n = _SCHUNKS[i]
    if n == _SCH:
      pltpu.sync_copy(rows, acc.at[idx], add=True)
    else:
      pltpu.sync_copy(rows.at[pl.ds(0, n)], acc.at[idx_t], add=True)
    if i + 1 < nchunks:
      pend = nxt

  plsc.subcore_barrier()
  pltpu.sync_copy(acc.at[pl.ds(s * _RPT, _RPT), :],
                  out_hbm.at[pl.ds(s * _RPT, _RPT), pl.ds(col, _CW)])


def _segment_sum_slab(ef, dst, zeros_tile, slab):
  mesh = plsc.VectorSubcoreMesh(core_axis_name="c", subcore_axis_name="s")
  f = pl.kernel(
      functools.partial(_scatter_body, slab),
      out_type=jax.ShapeDtypeStruct((_NPAD, D), jnp.float32),
      mesh=mesh,
      scratch_types=[
          pltpu.VMEM((_SCH,), jnp.int32),
          pltpu.VMEM((_SCH,), jnp.int32),
          pltpu.VMEM((_ST,), jnp.int32),
          pltpu.VMEM((_SCH, _CW), jnp.float32),
          pltpu.VMEM((_SCH, _CW), jnp.float32),
          pltpu.VMEM_SHARED((_NPAD, _CW), jnp.float32),
      ] + [pltpu.SemaphoreType.DMA] * 4,
  )
  return f(ef, dst, zeros_tile)


def _merged_body(slab_g, slab_s, pm_hbm, pg_hbm, src_hbm, dst_hbm,
                 ef_hbm, z_hbm, g1_hbm, g2_hbm, out_hbm,
                 idx_a, idx_b, ra0, rb0, ra1, rb1,
                 sidx0, sidx1, sidx_t, srows0, srows1, acc,
                 sa0, sb0, sa1, sb1, sw0, sw1, si0, si1, sr0, sr1):
  _gather_body(slab_g, pm_hbm, pg_hbm, src_hbm, dst_hbm, g1_hbm, g2_hbm,
               idx_a, idx_b, ra0, rb0, ra1, rb1, sa0, sb0, sa1, sb1, sw0, sw1)
  _scatter_body(slab_s, ef_hbm, dst_hbm, z_hbm, out_hbm,
                sidx0, sidx1, sidx_t, srows0, srows1, acc, si0, si1, sr0, sr1)


def _gather_and_segment_sum(pm, pg, src, dst, ef, zeros_tile, slab_g, slab_s):
  mesh = plsc.VectorSubcoreMesh(core_axis_name="c", subcore_axis_name="s")
  f = pl.kernel(
      functools.partial(_merged_body, slab_g, slab_s),
      out_type=[jax.ShapeDtypeStruct((ES, H // 2), jnp.int32),
                jax.ShapeDtypeStruct((ES, H // 2), jnp.int32),
                jax.ShapeDtypeStruct((_NPAD, D), jnp.float32)],
      mesh=mesh,
      scratch_types=[
          pltpu.VMEM((_EPW,), jnp.int32),
          pltpu.VMEM((_EPW,), jnp.int32),
          pltpu.VMEM((_GC, H // 2), jnp.int32),
          pltpu.VMEM((_GC, H // 2), jnp.int32),
          pltpu.VMEM((_GC, H // 2), jnp.int32),
          pltpu.VMEM((_GC, H // 2), jnp.int32),
          pltpu.VMEM((_SCH,), jnp.int32),
          pltpu.VMEM((_SCH,), jnp.int32),
          pltpu.VMEM((_ST,), jnp.int32),
          pltpu.VMEM((_SCH, _CW), jnp.float32),
          pltpu.VMEM((_SCH, _CW), jnp.float32),
          pltpu.VMEM_SHARED((_NPAD, _CW), jnp.float32),
      ] + [pltpu.SemaphoreType.DMA] * 10,
  )
  return f(pm, pg, src, dst, ef, zeros_tile)


def kernel(m2g_efeat, grid_nfeat, mesh_nfeat, src, dst,
           We1, be1, We2, be2, ge, bge,
           Wn1, bn1, Wn2, bn2, gn, bgn):
  a = We1[:D]
  wm = We1[D:2 * D]
  wg = We1[2 * D:]
  pm, pg = _project_nodes(mesh_nfeat, grid_nfeat, wm, wg)
  zeros_tile = jnp.zeros((_RPT, _CW), jnp.float32)
  efs = {}
  aggs = {}
  for slab in range(NSLAB):
    if slab < 2:
      g1, g2 = _gather_rows(pm, pg, src, dst, slab)
    else:
      g1, g2, aggs[slab - 2] = _gather_and_segment_sum(
          pm, pg, src, dst, efs[slab - 2], zeros_tile, slab, slab - 2)
    efs[slab] = _edge_mlp_slab(m2g_efeat, g1, g2, a, be1, We2, be2, ge, bge,
                               slab)
  for slab in (NSLAB - 2, NSLAB - 1):
    aggs[slab] = _segment_sum_slab(efs[slab], dst, zeros_tile, slab)
  return _node_mlp([aggs[i] for i in range(NSLAB)], grid_nfeat,
                   Wn1[:D], Wn1[D:], bn1, Wn2, bn2, gn, bgn)


# speedup vs baseline: 1.0085x; 1.0085x over previous
import functools

import jax
import jax.numpy as jnp
from jax import lax
from jax.experimental import pallas as pl
from jax.experimental.pallas import tpu as pltpu
from jax.experimental.pallas import tpu_sc as plsc

NC = 2
NS = 16
NW = NC * NS

N_GRID = 10000
N_MESH = 10000
E = 160000
D = 256
H = 256
LN_EPS = 1e-5

NSLAB = 5
ES = E // NSLAB
EBLK = 1000


def _round_f32_to_bf16_bits(x):
  b = lax.bitcast_convert_type(x, jnp.int32)
  rounded = b + jnp.int32(0x7FFF) + (lax.shift_right_logical(b, 16) & 1)
  return lax.shift_right_logical(rounded, 16)


def _pack_halves(y):
  lo = _round_f32_to_bf16_bits(y[:, :H // 2])
  hi = _round_f32_to_bf16_bits(y[:, H // 2:])
  return jnp.bitwise_or(lo, lax.shift_left(hi, 16))


def _unpack_halves(g):
  lo_f = lax.bitcast_convert_type(lax.shift_left(g, 16), jnp.float32)
  hi_f = lax.bitcast_convert_type(jnp.bitwise_and(g, jnp.int32(-65536)),
                                  jnp.float32)
  return jnp.concatenate([lo_f, hi_f], axis=-1)


def _proj_body(mesh_ref, grid_ref, wm_ref, wg_ref, pm_ref, pg_ref):
  pm = jnp.dot(mesh_ref[...], wm_ref[...], preferred_element_type=jnp.float32)
  pg = jnp.dot(grid_ref[...], wg_ref[...], preferred_element_type=jnp.float32)
  pm_ref[...] = _pack_halves(pm)
  pg_ref[...] = _pack_halves(pg)


def _project_nodes(mesh_nfeat, grid_nfeat, wm, wg):
  n = mesh_nfeat.shape[0]
  blk = 1000
  grid = (n // blk,)
  row_spec = pl.BlockSpec((blk, D), lambda i: (i, 0))
  w_spec = pl.BlockSpec((D, H), lambda i: (0, 0))
  return pl.pallas_call(
      _proj_body,
      grid=grid,
      in_specs=[row_spec, row_spec, w_spec, w_spec],
      out_specs=[pl.BlockSpec((blk, H // 2), lambda i: (i, 0))] * 2,
      out_shape=[jax.ShapeDtypeStruct((n, H // 2), jnp.int32)] * 2,
  )(mesh_nfeat, grid_nfeat, wm, wg)


def _layernorm_rows(y, g, b):
  m = jnp.mean(y, axis=-1, keepdims=True)
  yc = y - m
  v = jnp.mean(yc * yc, axis=-1, keepdims=True)
  return yc * lax.rsqrt(v + LN_EPS) * g + b


def _edge_body(x_ref, g1_ref, g2_ref, a_ref, b1_ref, w2_ref, b2_ref,
               g_ref, bg_ref, o_ref):
  x = jnp.dot(x_ref[...], a_ref[...], preferred_element_type=jnp.float32)
  x = x + _unpack_halves(g1_ref[...]) + _unpack_halves(g2_ref[...]) + b1_ref[...]
  h = x * jax.nn.sigmoid(x)
  y = jnp.dot(h, w2_ref[...], preferred_element_type=jnp.float32) + b2_ref[...]
  ef = _layernorm_rows(y, g_ref[...], bg_ref[...])
  o_ref[0, :, :] = ef[:, :D // 2]
  o_ref[1, :, :] = ef[:, D // 2:]


def _edge_mlp_slab(m2g_efeat, gath1, gath2, a, b1, w2, b2, g, bg, slab):
  grid = (ES // EBLK,)
  obl = slab * (ES // EBLK)
  w_spec = pl.BlockSpec((D, H), lambda i: (0, 0))
  v_spec = pl.BlockSpec((1, H), lambda i: (0, 0))
  return pl.pallas_call(
      _edge_body,
      grid=grid,
      in_specs=[pl.BlockSpec((EBLK, D), lambda i: (i + obl, 0)),
                pl.BlockSpec((EBLK, H // 2), lambda i: (i, 0)),
                pl.BlockSpec((EBLK, H // 2), lambda i: (i, 0)),
                w_spec, v_spec,
                pl.BlockSpec((H, D), lambda i: (0, 0)), v_spec,
                v_spec, v_spec],
      out_specs=pl.BlockSpec((2, EBLK, D // 2), lambda i: (0, i, 0)),
      out_shape=jax.ShapeDtypeStruct((2, ES, D // 2), jnp.float32),
  )(m2g_efeat, gath1, gath2, a, b1.reshape(1, H), w2, b2.reshape(1, D),
    g.reshape(1, D), bg.reshape(1, D))


def _node_body(a0_ref, a1_ref, a2_ref, a3_ref, a4_ref, gr_ref,
               w1a_ref, w1b_ref, b1_ref, w2_ref, b2_ref,
               g_ref, bg_ref, o_ref):
  agg = (a0_ref[...] + a1_ref[...] + a2_ref[...] + a3_ref[...] + a4_ref[...])
  x = (jnp.dot(agg, w1a_ref[...], preferred_element_type=jnp.float32)
       + jnp.dot(gr_ref[...], w1b_ref[...], preferred_element_type=jnp.float32)
       + b1_ref[...])
  h = x * jax.nn.sigmoid(x)
  y = jnp.dot(h, w2_ref[...], preferred_element_type=jnp.float32) + b2_ref[...]
  o_ref[...] = _layernorm_rows(y, g_ref[...], bg_ref[...]) + gr_ref[...]


def _node_mlp(aggs, grid_nfeat, w1a, w1b, b1, w2, b2, g, bg):
  blk = 1000
  grid = (N_GRID // blk,)
  row_spec = pl.BlockSpec((blk, D), lambda i: (i, 0))
  w_spec = pl.BlockSpec((D, H), lambda i: (0, 0))
  v_spec = pl.BlockSpec((1, H), lambda i: (0, 0))
  return pl.pallas_call(
      _node_body,
      grid=grid,
      in_specs=[row_spec] * NSLAB + [row_spec, w_spec, w_spec, v_spec,
                pl.BlockSpec((H, D), lambda i: (0, 0)), v_spec,
                v_spec, v_spec],
      out_specs=pl.BlockSpec((blk, D), lambda i: (i, 0)),
      out_shape=jax.ShapeDtypeStruct((N_GRID, D), jnp.float32),
  )(*aggs, grid_nfeat, w1a, w1b, b1.reshape(1, H), w2, b2.reshape(1, D),
    g.reshape(1, D), bg.reshape(1, D))


_EPW = ES // NW
_GC = 120
_GC_M = 56


def _chunk_list(total, c):
  n = total // c
  t = total - n * c
  return [c] * n + ([t] if t else [])


def _gather_body(slab, gc, pm_hbm, pg_hbm, src_hbm, dst_hbm, g1_hbm, g2_hbm,
                 idx_a, idx_b, ra0, rb0, ra1, rb1,
                 sa0, sb0, sa1, sb1, sw0, sw1):
  chunks = _chunk_list(_EPW, gc)
  wid = lax.axis_index("s") * NC + lax.axis_index("c")
  lbase = wid * _EPW
  base = slab * ES + lbase
  pltpu.sync_copy(src_hbm.at[pl.ds(base, _EPW)], idx_a)
  pltpu.sync_copy(dst_hbm.at[pl.ds(base, _EPW)], idx_b)

  bufs = [(ra0, rb0, sa0, sb0, sw0), (ra1, rb1, sa1, sb1, sw1)]
  nchunks = len(chunks)

  def issue(i):
    ra, rb, sa, sb, _ = bufs[i % 2]
    off = i * gc
    n = chunks[i]
    ca = pltpu.async_copy(pm_hbm.at[idx_a.at[pl.ds(off, n)]],
                          ra.at[pl.ds(0, n)], sa)
    cb = pltpu.async_copy(pg_hbm.at[idx_b.at[pl.ds(off, n)]],
                          rb.at[pl.ds(0, n)], sb)
    return ca, cb

  def issue_writes(i):
    ra, rb, _, _, sw = bufs[i % 2]
    off = i * gc
    n = chunks[i]
    wa = pltpu.async_copy(ra.at[pl.ds(0, n)],
                          g1_hbm.at[pl.ds(lbase + off, n), :], sw)
    wb = pltpu.async_copy(rb.at[pl.ds(0, n)],
                          g2_hbm.at[pl.ds(lbase + off, n), :], sw)
    return wa, wb

  pend_w = [None, None]
  g = issue(0)
  for i in range(nchunks):
    if i + 1 < nchunks:
      if pend_w[(i + 1) % 2] is not None:
        for wdesc in pend_w[(i + 1) % 2]:
          wdesc.wait()
        pend_w[(i + 1) % 2] = None
      g_next = issue(i + 1)
    for gdesc in g:
      gdesc.wait()
    pend_w[i % 2] = issue_writes(i)
    if i + 1 < nchunks:
      g = g_next
  for p in pend_w:
    if p is not None:
      for wdesc in p:
        wdesc.wait()


def _gather_rows(pm, pg, src, dst, slab):
  mesh = plsc.VectorSubcoreMesh(core_axis_name="c", subcore_axis_name="s")
  f = pl.kernel(
      functools.partial(_gather_body, slab, _GC),
      out_type=[jax.ShapeDtypeStruct((ES, H // 2), jnp.int32)] * 2,
      mesh=mesh,
      scratch_types=[
          pltpu.VMEM((_EPW,), jnp.int32),
          pltpu.VMEM((_EPW,), jnp.int32),
          pltpu.VMEM((_GC, H // 2), jnp.int32),
          pltpu.VMEM((_GC, H // 2), jnp.int32),
          pltpu.VMEM((_GC, H // 2), jnp.int32),
          pltpu.VMEM((_GC, H // 2), jnp.int32),
      ] + [pltpu.SemaphoreType.DMA] * 6,
  )
  return f(pm, pg, src, dst)


_CW = D // NC
_EPT = ES // NS
_SCH = 128
_SCH_M = 64
_NPAD = 10240
_RPT = _NPAD // NS


def _scatter_tail(total, c):
  t = total % c
  return t if t else c


def _scatter_body(slab, sch, ef_hbm, dst_hbm, z_hbm, out_hbm,
                  idx0, idx1, idx_t, rows0, rows1, acc, si0, si1, sr0, sr1):
  chunks = _chunk_list(_EPT, sch)
  c = lax.axis_index("c")
  s = lax.axis_index("s")
  col = c * _CW
  sbase = slab * ES
  bufs = [(idx0, rows0, si0, sr0), (idx1, rows1, si1, sr1)]
  nchunks = len(chunks)

  def issue(i):
    idx, rows, si, sr = bufs[i % 2]
    off = s * _EPT + i * sch
    n = chunks[i]
    idx_dst = idx if n == sch else idx_t
    ci = pltpu.async_copy(dst_hbm.at[pl.ds(sbase + off, n)], idx_dst, si)
    cr = pltpu.async_copy(ef_hbm.at[c, pl.ds(off, n), :],
                          rows.at[pl.ds(0, n)], sr)
    return ci, cr

  pend = issue(0)
  pltpu.sync_copy(z_hbm, acc.at[pl.ds(s * _RPT, _RPT), :])
  plsc.subcore_barrier()

  for i in range(nchunks):
    if i + 1 < nchunks:
      nxt = issue(i + 1)
    for d in pend:
      d.wait()
    idx, rows, _, _ = bufs[i % 2]
    n = chunks[i]
    if n == sch:
      pltpu.sync_copy(rows, acc.at[idx], add=True)
    else:
      pltpu.sync_copy(rows.at[pl.ds(0, n)], acc.at[idx_t], add=True)
    if i + 1 < nchunks:
      pend = nxt

  plsc.subcore_barrier()
  pltpu.sync_copy(acc.at[pl.ds(s * _RPT, _RPT), :],
                  out_hbm.at[pl.ds(s * _RPT, _RPT), pl.ds(col, _CW)])


def _segment_sum_slab(ef, dst, zeros_tile, slab):
  mesh = plsc.VectorSubcoreMesh(core_axis_name="c", subcore_axis_name="s")
  f = pl.kernel(
      functools.partial(_scatter_body, slab, _SCH),
      out_type=jax.ShapeDtypeStruct((_NPAD, D), jnp.float32),
      mesh=mesh,
      scratch_types=[
          pltpu.VMEM((_SCH,), jnp.int32),
          pltpu.VMEM((_SCH,), jnp.int32),
          pltpu.VMEM((_scatter_tail(_EPT, _SCH),), jnp.int32),
          pltpu.VMEM((_SCH, _CW), jnp.float32),
          pltpu.VMEM((_SCH, _CW), jnp.float32),
          pltpu.VMEM_SHARED((_NPAD, _CW), jnp.float32),
      ] + [pltpu.SemaphoreType.DMA] * 4,
  )
  return f(ef, dst, zeros_tile)


def _merged_body(slab_g, slab_s, pm_hbm, pg_hbm, src_hbm, dst_hbm,
                 ef_hbm, z_hbm, g1_hbm, g2_hbm, out_hbm,
                 idx_a, idx_b, ra0, rb0, ra1, rb1,
                 sidx0, sidx1, sidx_t, srows0, srows1, acc,
                 sa0, sb0, sa1, sb1, sw0, sw1, si0, si1, sr0, sr1):
  _gather_body(slab_g, _GC_M, pm_hbm, pg_hbm, src_hbm, dst_hbm, g1_hbm,
               g2_hbm, idx_a, idx_b, ra0, rb0, ra1, rb1,
               sa0, sb0, sa1, sb1, sw0, sw1)
  _scatter_body(slab_s, _SCH_M, ef_hbm, dst_hbm, z_hbm, out_hbm,
                sidx0, sidx1, sidx_t, srows0, srows1, acc, si0, si1, sr0, sr1)


def _gather_and_segment_sum(pm, pg, src, dst, ef, zeros_tile, slab_g, slab_s):
  mesh = plsc.VectorSubcoreMesh(core_axis_name="c", subcore_axis_name="s")
  f = pl.kernel(
      functools.partial(_merged_body, slab_g, slab_s),
      out_type=[jax.ShapeDtypeStruct((ES, H // 2), jnp.int32),
                jax.ShapeDtypeStruct((ES, H // 2), jnp.int32),
                jax.ShapeDtypeStruct((_NPAD, D), jnp.float32)],
      mesh=mesh,
      scratch_types=[
          pltpu.VMEM((_EPW,), jnp.int32),
          pltpu.VMEM((_EPW,), jnp.int32),
          pltpu.VMEM((_GC_M, H // 2), jnp.int32),
          pltpu.VMEM((_GC_M, H // 2), jnp.int32),
          pltpu.VMEM((_GC_M, H // 2), jnp.int32),
          pltpu.VMEM((_GC_M, H // 2), jnp.int32),
          pltpu.VMEM((_SCH_M,), jnp.int32),
          pltpu.VMEM((_SCH_M,), jnp.int32),
          pltpu.VMEM((_scatter_tail(_EPT, _SCH_M),), jnp.int32),
          pltpu.VMEM((_SCH_M, _CW), jnp.float32),
          pltpu.VMEM((_SCH_M, _CW), jnp.float32),
          pltpu.VMEM_SHARED((_NPAD, _CW), jnp.float32),
      ] + [pltpu.SemaphoreType.DMA] * 10,
  )
  return f(pm, pg, src, dst, ef, zeros_tile)


def kernel(m2g_efeat, grid_nfeat, mesh_nfeat, src, dst,
           We1, be1, We2, be2, ge, bge,
           Wn1, bn1, Wn2, bn2, gn, bgn):
  a = We1[:D]
  wm = We1[D:2 * D]
  wg = We1[2 * D:]
  pm, pg = _project_nodes(mesh_nfeat, grid_nfeat, wm, wg)
  zeros_tile = jnp.zeros((_RPT, _CW), jnp.float32)
  efs = {}
  aggs = {}
  for slab in range(NSLAB):
    if slab < 2:
      g1, g2 = _gather_rows(pm, pg, src, dst, slab)
    else:
      g1, g2, aggs[slab - 2] = _gather_and_segment_sum(
          pm, pg, src, dst, efs[slab - 2], zeros_tile, slab, slab - 2)
    efs[slab] = _edge_mlp_slab(m2g_efeat, g1, g2, a, be1, We2, be2, ge, bge,
                               slab)
  for slab in (NSLAB - 2, NSLAB - 1):
    aggs[slab] = _segment_sum_slab(efs[slab], dst, zeros_tile, slab)
  return _node_mlp([aggs[i] for i in range(NSLAB)], grid_nfeat,
                   Wn1[:D], Wn1[D:], bn1, Wn2, bn2, gn, bgn)


# speedup vs baseline: 1.0667x; 1.0578x over previous
import functools

import jax
import jax.numpy as jnp
from jax import lax
from jax.experimental import pallas as pl
from jax.experimental.pallas import tpu as pltpu
from jax.experimental.pallas import tpu_sc as plsc

NC = 2
NS = 16
NW = NC * NS

N_GRID = 10000
N_MESH = 10000
E = 160000
D = 256
H = 256
LN_EPS = 1e-5

NSLAB = 5
ES = E // NSLAB
EBLK = 1000


def _round_f32_to_bf16_bits(x):
  b = lax.bitcast_convert_type(x, jnp.int32)
  rounded = b + jnp.int32(0x7FFF) + (lax.shift_right_logical(b, 16) & 1)
  return lax.shift_right_logical(rounded, 16)


def _pack_halves(y):
  lo = _round_f32_to_bf16_bits(y[:, :H // 2])
  hi = _round_f32_to_bf16_bits(y[:, H // 2:])
  return jnp.bitwise_or(lo, lax.shift_left(hi, 16))


def _unpack_halves(g):
  lo_f = lax.bitcast_convert_type(lax.shift_left(g, 16), jnp.float32)
  hi_f = lax.bitcast_convert_type(jnp.bitwise_and(g, jnp.int32(-65536)),
                                  jnp.float32)
  return jnp.concatenate([lo_f, hi_f], axis=-1)


def _proj_body(mesh_ref, grid_ref, wm_ref, wg_ref, pm_ref, pg_ref):
  pm = jnp.dot(mesh_ref[...], wm_ref[...], preferred_element_type=jnp.float32)
  pg = jnp.dot(grid_ref[...], wg_ref[...], preferred_element_type=jnp.float32)
  pm_ref[...] = _pack_halves(pm)
  pg_ref[...] = _pack_halves(pg)


def _project_nodes(mesh_nfeat, grid_nfeat, wm, wg):
  n = mesh_nfeat.shape[0]
  blk = 1000
  grid = (n // blk,)
  row_spec = pl.BlockSpec((blk, D), lambda i: (i, 0))
  w_spec = pl.BlockSpec((D, H), lambda i: (0, 0))
  return pl.pallas_call(
      _proj_body,
      grid=grid,
      in_specs=[row_spec, row_spec, w_spec, w_spec],
      out_specs=[pl.BlockSpec((blk, H // 2), lambda i: (i, 0))] * 2,
      out_shape=[jax.ShapeDtypeStruct((n, H // 2), jnp.int32)] * 2,
  )(mesh_nfeat, grid_nfeat, wm, wg)


def _layernorm_rows(y, g, b):
  m = jnp.mean(y, axis=-1, keepdims=True)
  yc = y - m
  v = jnp.mean(yc * yc, axis=-1, keepdims=True)
  return yc * lax.rsqrt(v + LN_EPS) * g + b


def _edge_body(x_ref, g1_ref, g2_ref, a_ref, b1_ref, w2_ref, b2_ref,
               g_ref, bg_ref, o_ref):
  x = jnp.dot(x_ref[...], a_ref[...], preferred_element_type=jnp.float32)
  x = x + _unpack_halves(g1_ref[...]) + _unpack_halves(g2_ref[...]) + b1_ref[...]
  h = x * jax.nn.sigmoid(x)
  y = jnp.dot(h, w2_ref[...], preferred_element_type=jnp.float32) + b2_ref[...]
  ef = _layernorm_rows(y, g_ref[...], bg_ref[...])
  o_ref[0, :, :] = ef[:, :D // 2]
  o_ref[1, :, :] = ef[:, D // 2:]


def _edge_mlp_slab(m2g_efeat, gath1, gath2, a, b1, w2, b2, g, bg, slab):
  grid = (ES // EBLK,)
  obl = slab * (ES // EBLK)
  w_spec = pl.BlockSpec((D, H), lambda i: (0, 0))
  v_spec = pl.BlockSpec((1, H), lambda i: (0, 0))
  return pl.pallas_call(
      _edge_body,
      grid=grid,
      in_specs=[pl.BlockSpec((EBLK, D), lambda i: (i + obl, 0)),
                pl.BlockSpec((EBLK, H // 2), lambda i: (i, 0)),
                pl.BlockSpec((EBLK, H // 2), lambda i: (i, 0)),
                w_spec, v_spec,
                pl.BlockSpec((H, D), lambda i: (0, 0)), v_spec,
                v_spec, v_spec],
      out_specs=pl.BlockSpec((2, EBLK, D // 2), lambda i: (0, i, 0)),
      out_shape=jax.ShapeDtypeStruct((2, ES, D // 2), jnp.float32),
  )(m2g_efeat, gath1, gath2, a, b1.reshape(1, H), w2, b2.reshape(1, D),
    g.reshape(1, D), bg.reshape(1, D))


def _node_body(a0_ref, a1_ref, a2_ref, a3_ref, a4_ref, gr_ref,
               w1a_ref, w1b_ref, b1_ref, w2_ref, b2_ref,
               g_ref, bg_ref, o_ref):
  agg = (a0_ref[...] + a1_ref[...] + a2_ref[...] + a3_ref[...] + a4_ref[...])
  x = (jnp.dot(agg, w1a_ref[...], preferred_element_type=jnp.float32)
       + jnp.dot(gr_ref[...], w1b_ref[...], preferred_element_type=jnp.float32)
       + b1_ref[...])
  h = x * jax.nn.sigmoid(x)
  y = jnp.dot(h, w2_ref[...], preferred_element_type=jnp.float32) + b2_ref[...]
  o_ref[...] = _layernorm_rows(y, g_ref[...], bg_ref[...]) + gr_ref[...]


def _node_mlp(aggs, grid_nfeat, w1a, w1b, b1, w2, b2, g, bg):
  blk = 1000
  grid = (N_GRID // blk,)
  row_spec = pl.BlockSpec((blk, D), lambda i: (i, 0))
  w_spec = pl.BlockSpec((D, H), lambda i: (0, 0))
  v_spec = pl.BlockSpec((1, H), lambda i: (0, 0))
  return pl.pallas_call(
      _node_body,
      grid=grid,
      in_specs=[row_spec] * NSLAB + [row_spec, w_spec, w_spec, v_spec,
                pl.BlockSpec((H, D), lambda i: (0, 0)), v_spec,
                v_spec, v_spec],
      out_specs=pl.BlockSpec((blk, D), lambda i: (i, 0)),
      out_shape=jax.ShapeDtypeStruct((N_GRID, D), jnp.float32),
  )(*aggs, grid_nfeat, w1a, w1b, b1.reshape(1, H), w2, b2.reshape(1, D),
    g.reshape(1, D), bg.reshape(1, D))


_EPW = ES // NW
_GC = 120


def _chunk_list(total, c):
  n = total // c
  t = total - n * c
  return [c] * n + ([t] if t else [])


def _gather_body(slab, gc, pm_hbm, pg_hbm, src_hbm, dst_hbm, g1_hbm, g2_hbm,
                 idx_a, idx_b, ra0, rb0, ra1, rb1,
                 sa0, sb0, sa1, sb1, sw0, sw1):
  chunks = _chunk_list(_EPW, gc)
  wid = lax.axis_index("s") * NC + lax.axis_index("c")
  lbase = wid * _EPW
  base = slab * ES + lbase
  pltpu.sync_copy(src_hbm.at[pl.ds(base, _EPW)], idx_a)
  pltpu.sync_copy(dst_hbm.at[pl.ds(base, _EPW)], idx_b)

  bufs = [(ra0, rb0, sa0, sb0, sw0), (ra1, rb1, sa1, sb1, sw1)]
  nchunks = len(chunks)

  def issue(i):
    ra, rb, sa, sb, _ = bufs[i % 2]
    off = i * gc
    n = chunks[i]
    ca = pltpu.async_copy(pm_hbm.at[idx_a.at[pl.ds(off, n)]],
                          ra.at[pl.ds(0, n)], sa)
    cb = pltpu.async_copy(pg_hbm.at[idx_b.at[pl.ds(off, n)]],
                          rb.at[pl.ds(0, n)], sb)
    return ca, cb

  def issue_writes(i):
    ra, rb, _, _, sw = bufs[i % 2]
    off = i * gc
    n = chunks[i]
    wa = pltpu.async_copy(ra.at[pl.ds(0, n)],
                          g1_hbm.at[pl.ds(lbase + off, n), :], sw)
    wb = pltpu.async_copy(rb.at[pl.ds(0, n)],
                          g2_hbm.at[pl.ds(lbase + off, n), :], sw)
    return wa, wb

  pend_w = [None, None]
  g = issue(0)
  for i in range(nchunks):
    if i + 1 < nchunks:
      if pend_w[(i + 1) % 2] is not None:
        for wdesc in pend_w[(i + 1) % 2]:
          wdesc.wait()
        pend_w[(i + 1) % 2] = None
      g_next = issue(i + 1)
    for gdesc in g:
      gdesc.wait()
    pend_w[i % 2] = issue_writes(i)
    if i + 1 < nchunks:
      g = g_next
  for p in pend_w:
    if p is not None:
      for wdesc in p:
        wdesc.wait()


def _gather_rows(pm, pg, src, dst, slab):
  mesh = plsc.VectorSubcoreMesh(core_axis_name="c", subcore_axis_name="s")
  f = pl.kernel(
      functools.partial(_gather_body, slab, _GC),
      out_type=[jax.ShapeDtypeStruct((ES, H // 2), jnp.int32)] * 2,
      mesh=mesh,
      scratch_types=[
          pltpu.VMEM((_EPW,), jnp.int32),
          pltpu.VMEM((_EPW,), jnp.int32),
          pltpu.VMEM((_GC, H // 2), jnp.int32),
          pltpu.VMEM((_GC, H // 2), jnp.int32),
          pltpu.VMEM((_GC, H // 2), jnp.int32),
          pltpu.VMEM((_GC, H // 2), jnp.int32),
      ] + [pltpu.SemaphoreType.DMA] * 6,
  )
  return f(pm, pg, src, dst)


_CW = D // NC
_EPT = ES // NS
_SCH = 128
_NPAD = 10240
_RPT = _NPAD // NS


def _scatter_tail(total, c):
  t = total % c
  return t if t else c


def _scatter_body(slab, sch, ef_hbm, dst_hbm, z_hbm, out_hbm,
                  idx0, idx1, idx_t, rows0, rows1, acc, si0, si1, sr0, sr1):
  chunks = _chunk_list(_EPT, sch)
  c = lax.axis_index("c")
  s = lax.axis_index("s")
  col = c * _CW
  sbase = slab * ES
  bufs = [(idx0, rows0, si0, sr0), (idx1, rows1, si1, sr1)]
  nchunks = len(chunks)

  def issue(i):
    idx, rows, si, sr = bufs[i % 2]
    off = s * _EPT + i * sch
    n = chunks[i]
    idx_dst = idx if n == sch else idx_t
    ci = pltpu.async_copy(dst_hbm.at[pl.ds(sbase + off, n)], idx_dst, si)
    cr = pltpu.async_copy(ef_hbm.at[c, pl.ds(off, n), :],
                          rows.at[pl.ds(0, n)], sr)
    return ci, cr

  pend = issue(0)
  pltpu.sync_copy(z_hbm, acc.at[pl.ds(s * _RPT, _RPT), :])
  plsc.subcore_barrier()

  for i in range(nchunks):
    if i + 1 < nchunks:
      nxt = issue(i + 1)
    for d in pend:
      d.wait()
    idx, rows, _, _ = bufs[i % 2]
    n = chunks[i]
    if n == sch:
      pltpu.sync_copy(rows, acc.at[idx], add=True)
    else:
      pltpu.sync_copy(rows.at[pl.ds(0, n)], acc.at[idx_t], add=True)
    if i + 1 < nchunks:
      pend = nxt

  plsc.subcore_barrier()
  pltpu.sync_copy(acc.at[pl.ds(s * _RPT, _RPT), :],
                  out_hbm.at[pl.ds(s * _RPT, _RPT), pl.ds(col, _CW)])


def _segment_sum_slab(ef, dst, zeros_tile, slab):
  mesh = plsc.VectorSubcoreMesh(core_axis_name="c", subcore_axis_name="s")
  f = pl.kernel(
      functools.partial(_scatter_body, slab, _SCH),
      out_type=jax.ShapeDtypeStruct((_NPAD, D), jnp.float32),
      mesh=mesh,
      scratch_types=[
          pltpu.VMEM((_SCH,), jnp.int32),
          pltpu.VMEM((_SCH,), jnp.int32),
          pltpu.VMEM((_scatter_tail(_EPT, _SCH),), jnp.int32),
          pltpu.VMEM((_SCH, _CW), jnp.float32),
          pltpu.VMEM((_SCH, _CW), jnp.float32),
          pltpu.VMEM_SHARED((_NPAD, _CW), jnp.float32),
      ] + [pltpu.SemaphoreType.DMA] * 4,
  )
  return f(ef, dst, zeros_tile)


def kernel(m2g_efeat, grid_nfeat, mesh_nfeat, src, dst,
           We1, be1, We2, be2, ge, bge,
           Wn1, bn1, Wn2, bn2, gn, bgn):
  a = We1[:D]
  wm = We1[D:2 * D]
  wg = We1[2 * D:]
  pm, pg = _project_nodes(mesh_nfeat, grid_nfeat, wm, wg)
  zeros_tile = jnp.zeros((_RPT, _CW), jnp.float32)
  aggs = []
  for slab in range(NSLAB):
    g1, g2 = _gather_rows(pm, pg, src, dst, slab)
    ef = _edge_mlp_slab(m2g_efeat, g1, g2, a, be1, We2, be2, ge, bge, slab)
    aggs.append(_segment_sum_slab(ef, dst, zeros_tile, slab))
  return _node_mlp(aggs, grid_nfeat, Wn1[:D], Wn1[D:], bn1,
                   Wn2, bn2, gn, bgn)
